# baseline (speedup 1.0000x reference)
import functools
import math

import numpy as np
import jax
import jax.numpy as jnp
from jax.experimental import pallas as pl
from jax.experimental.pallas import tpu as pltpu

LEAKY_SLOPE = 0.01
VMEM_LIMIT_BYTES = 60 * 1024 * 1024


def _pool_matrix(H, W, s):
    P = np.zeros((s * s, H * W), np.float32)
    for i in range(s):
        r0, r1 = (i * H) // s, -((-(i + 1) * H) // s)
        for j in range(s):
            c0, c1 = (j * W) // s, -((-(j + 1) * W) // s)
            val = 1.0 / ((r1 - r0) * (c1 - c0))
            for rr in range(r0, r1):
                for cc in range(c0, c1):
                    P[i * s + j, rr * W + cc] = val
    return P


def _up_matrix(H, W, s):
    def axis_w(out_len, in_len):
        M = np.zeros((out_len, in_len), np.float32)
        for o in range(out_len):
            if in_len == 1:
                M[o, 0] = 1.0
                continue
            src = o * (in_len - 1) / (out_len - 1)
            i0 = min(int(math.floor(src)), in_len - 1)
            i1 = min(i0 + 1, in_len - 1)
            f = src - i0
            M[o, i0] += 1.0 - f
            M[o, i1] += f
        return M
    Wy, Wx = axis_w(H, s), axis_w(W, s)
    return np.einsum('yi,xj->yxij', Wy, Wx).reshape(H * W, s * s).astype(np.float32)


@functools.lru_cache(maxsize=None)
def _stage_constants(H, W, sizes, cout):
    s_tot = sum(s * s for s in sizes)
    P = np.zeros((s_tot, H * W), np.float32)
    B = np.zeros((H * W, s_tot), np.float32)
    M = np.zeros((s_tot, len(sizes) * cout), np.float32)
    off = 0
    for si, s in enumerate(sizes):
        P[off:off + s * s, :] = _pool_matrix(H, W, s)
        B[:, off:off + s * s] = _up_matrix(H, W, s)
        M[off:off + s * s, si * cout:(si + 1) * cout] = 1.0
        off += s * s
    return (jnp.asarray(P, jnp.bfloat16), jnp.asarray(B, jnp.bfloat16),
            jnp.asarray(M))


def _psp_kernel(x_ref, p_ref, m_ref, up_ref,
                w0_ref, w1_ref, w2_ref, w3_ref,
                s0_ref, s1_ref, s2_ref, s3_ref,
                b0_ref, b1_ref, b2_ref, b3_ref,
                w9_ref, bs_ref, bb_ref,
                o_ref, acc_ref, wbig_ref, wcat_ref):
    _, H, W, Co = o_ref.shape
    nCo = wcat_ref.shape[1]

    @pl.when(pl.program_id(1) == 0)
    def _prepare_weights():
        bs = bs_ref[...]
        for t in range(9):
            wbig_ref[:, t * Co:(t + 1) * Co] = (
                w9_ref[t] * bs).astype(jnp.bfloat16)
        for i, (w_r, s_r) in enumerate(
                zip((w0_ref, w1_ref, w2_ref, w3_ref),
                    (s0_ref, s1_ref, s2_ref, s3_ref))):
            wcat_ref[:, i * Co:(i + 1) * Co] = (
                w_r[...] * s_r[...]).astype(jnp.bfloat16)

    x16 = x_ref[0].astype(jnp.bfloat16)

    pooled = jnp.dot(p_ref[...], x16, preferred_element_type=jnp.float32)
    y = jnp.dot(pooled.astype(jnp.bfloat16), wcat_ref[...],
                preferred_element_type=jnp.float32)
    y = y + jnp.concatenate(
        [b0_ref[...], b1_ref[...], b2_ref[...], b3_ref[...]], axis=1)
    y = jnp.where(y >= 0, y, LEAKY_SLOPE * y) * m_ref[...]

    z = jnp.dot(y.astype(jnp.bfloat16), wbig_ref[0:nCo, :],
                preferred_element_type=jnp.float32)
    upt = jnp.dot(up_ref[...], z.astype(jnp.bfloat16),
                  preferred_element_type=jnp.float32)

    G = 3 * Co
    first = True
    for dy in (1, 0, 2):
        tg = jnp.dot(x16, wbig_ref[nCo:, dy * G:(dy + 1) * G],
                     preferred_element_type=jnp.float32)
        tg = tg + upt[:, dy * G:(dy + 1) * G]
        for dx in ((1, 0, 2) if dy == 1 else (0, 1, 2)):
            tap = tg[:, dx * Co:(dx + 1) * Co].reshape(H, W, Co)
            if first:
                acc_ref[pl.ds(2 - dy, H), pl.ds(2 - dx, W), :] = tap
                first = False
            else:
                acc_ref[pl.ds(2 - dy, H), pl.ds(2 - dx, W), :] += tap
    out = acc_ref[pl.ds(1, H), pl.ds(1, W), :] + bb_ref[0]
    o_ref[0] = jnp.where(out >= 0, out, LEAKY_SLOPE * out)


def kernel(feats, w0, scale0, bias0, w1, scale1, bias1, w2, scale2, bias2,
           w3, scale3, bias3, wb, b_scale, b_bias):
    sizes = (1, 2, 3, 6)
    N, Cin, H, W = feats.shape
    HW = H * W
    Cout = w0.shape[1]
    nCo = len(sizes) * Cout
    S = sum(s * s for s in sizes)
    Ct = nCo + Cin

    x_flat = jnp.transpose(feats, (0, 2, 3, 1)).reshape(N, HW, Cin)
    wb9 = wb.reshape(9, Ct, Cout)

    p_all, b_all, mask = _stage_constants(H, W, sizes, Cout)

    flops = 2 * N * (S * HW * Cin + S * Cin * nCo + S * nCo * 9 * Cout
                     + HW * S * 9 * Cout + 9 * HW * Cin * Cout)
    bytes_accessed = (4 * N * HW * Cin + 2 * S * HW + 2 * HW * S
                      + 4 * Ct * 9 * Cout + 4 * Cin * nCo
                      + 4 * N * HW * Cout)

    NB = 2
    NI = N // NB
    zero2 = lambda i, j: (0, 0)
    zero3 = lambda i, j: (0, 0, 0)

    out_nhwc = pl.pallas_call(
        _psp_kernel,
        out_shape=jax.ShapeDtypeStruct((N, H, W, Cout), jnp.float32),
        grid=(NB, NI),
        in_specs=[
            pl.BlockSpec((1, HW, Cin), lambda i, j: (i * NI + j, 0, 0)),
            pl.BlockSpec((S, HW), zero2),
            pl.BlockSpec((S, nCo), zero2),
            pl.BlockSpec((HW, S), zero2),
        ] + [pl.BlockSpec((Cin, Cout), zero2)] * 4
          + [pl.BlockSpec((1, Cout), zero2)] * 8
          + [
            pl.BlockSpec((9, Ct, Cout), zero3),
            pl.BlockSpec((1, Cout), zero2),
            pl.BlockSpec((1, Cout), zero2),
        ],
        out_specs=pl.BlockSpec((1, H, W, Cout), lambda i, j: (i * NI + j, 0, 0, 0)),
        scratch_shapes=[
            pltpu.VMEM((H + 2, W + 2, Cout), jnp.float32),
            pltpu.VMEM((Ct, 9 * Cout), jnp.bfloat16),
            pltpu.VMEM((Cin, nCo), jnp.bfloat16),
        ],
        compiler_params=pltpu.CompilerParams(
            dimension_semantics=("parallel", "arbitrary"),
            vmem_limit_bytes=VMEM_LIMIT_BYTES),
        cost_estimate=pl.CostEstimate(flops=flops, transcendentals=0,
                                      bytes_accessed=bytes_accessed),
    )(x_flat, p_all, mask, b_all,
      w0, w1, w2, w3, scale0, scale1, scale2, scale3,
      bias0, bias1, bias2, bias3, wb9, b_scale, b_bias)

    return jnp.transpose(out_nhwc, (0, 3, 1, 2))


# speedup vs baseline: 1.1569x; 1.1569x over previous
import functools
import math

import numpy as np
import jax
import jax.numpy as jnp
from jax.experimental import pallas as pl
from jax.experimental.pallas import tpu as pltpu

LEAKY_SLOPE = 0.01
VMEM_LIMIT_BYTES = 60 * 1024 * 1024


def _pool_matrix(H, W, s):
    P = np.zeros((s * s, H * W), np.float32)
    for i in range(s):
        r0, r1 = (i * H) // s, -((-(i + 1) * H) // s)
        for j in range(s):
            c0, c1 = (j * W) // s, -((-(j + 1) * W) // s)
            val = 1.0 / ((r1 - r0) * (c1 - c0))
            for rr in range(r0, r1):
                for cc in range(c0, c1):
                    P[i * s + j, rr * W + cc] = val
    return P


def _up_matrix(H, W, s):
    def axis_w(out_len, in_len):
        M = np.zeros((out_len, in_len), np.float32)
        for o in range(out_len):
            if in_len == 1:
                M[o, 0] = 1.0
                continue
            src = o * (in_len - 1) / (out_len - 1)
            i0 = min(int(math.floor(src)), in_len - 1)
            i1 = min(i0 + 1, in_len - 1)
            f = src - i0
            M[o, i0] += 1.0 - f
            M[o, i1] += f
        return M
    Wy, Wx = axis_w(H, s), axis_w(W, s)
    return np.einsum('yi,xj->yxij', Wy, Wx).reshape(H * W, s * s).astype(np.float32)


@functools.lru_cache(maxsize=None)
def _stage_constants(H, W, sizes, cout):
    s_tot = sum(s * s for s in sizes)
    P = np.zeros((s_tot, H * W), np.float32)
    B = np.zeros((H * W, s_tot), np.float32)
    M = np.zeros((s_tot, len(sizes) * cout), np.float32)
    off = 0
    for si, s in enumerate(sizes):
        P[off:off + s * s, :] = _pool_matrix(H, W, s)
        B[:, off:off + s * s] = _up_matrix(H, W, s)
        M[off:off + s * s, si * cout:(si + 1) * cout] = 1.0
        off += s * s
    w_idx = np.arange(H * W) % W
    mnot_first = (w_idx != 0).astype(np.float32).reshape(H * W, 1)
    mnot_last = (w_idx != W - 1).astype(np.float32).reshape(H * W, 1)
    return (jnp.asarray(P, jnp.bfloat16), jnp.asarray(B, jnp.bfloat16),
            jnp.asarray(M), jnp.asarray(mnot_first), jnp.asarray(mnot_last))


def _psp_kernel(x_ref, p_ref, m_ref, up_ref, mf_ref, ml_ref,
                w0_ref, w1_ref, w2_ref, w3_ref,
                s0_ref, s1_ref, s2_ref, s3_ref,
                b0_ref, b1_ref, b2_ref, b3_ref,
                w9_ref, bs_ref, bb_ref,
                o_ref, acc_ref, wbig_ref, wcat_ref):
    _, H, W, Co = o_ref.shape
    nCo = wcat_ref.shape[1]

    @pl.when(pl.program_id(1) == 0)
    def _prepare_weights():
        bs = bs_ref[...]
        for t in range(9):
            wbig_ref[:, t * Co:(t + 1) * Co] = (
                w9_ref[t] * bs).astype(jnp.bfloat16)
        for i, (w_r, s_r) in enumerate(
                zip((w0_ref, w1_ref, w2_ref, w3_ref),
                    (s0_ref, s1_ref, s2_ref, s3_ref))):
            wcat_ref[:, i * Co:(i + 1) * Co] = (
                w_r[...] * s_r[...]).astype(jnp.bfloat16)

    x16 = x_ref[0].astype(jnp.bfloat16)

    pooled = jnp.dot(p_ref[...], x16, preferred_element_type=jnp.float32)
    y = jnp.dot(pooled.astype(jnp.bfloat16), wcat_ref[...],
                preferred_element_type=jnp.float32)
    y = y + jnp.concatenate(
        [b0_ref[...], b1_ref[...], b2_ref[...], b3_ref[...]], axis=1)
    y = jnp.where(y >= 0, y, LEAKY_SLOPE * y) * m_ref[...]

    z = jnp.dot(y.astype(jnp.bfloat16), wbig_ref[0:nCo, :],
                preferred_element_type=jnp.float32)
    taps = (jnp.dot(up_ref[...], z.astype(jnp.bfloat16),
                    preferred_element_type=jnp.float32) +
            jnp.dot(x16, wbig_ref[nCo:, :],
                    preferred_element_type=jnp.float32))

    zrow = jnp.zeros((1, Co), jnp.float32)
    acc_ref[...] = jnp.zeros_like(acc_ref)
    for dy in range(3):
        t0 = taps[:, (dy * 3 + 0) * Co:(dy * 3 + 1) * Co]
        t1 = taps[:, (dy * 3 + 1) * Co:(dy * 3 + 2) * Co]
        t2 = taps[:, (dy * 3 + 2) * Co:(dy * 3 + 3) * Co]
        sh0 = jnp.concatenate([zrow, t0[:-1]], axis=0) * mf_ref[...]
        sh2 = jnp.concatenate([t2[1:], zrow], axis=0) * ml_ref[...]
        row = t1 + sh0 + sh2
        acc_ref[pl.ds(2 - dy, H), :, :] += row.reshape(H, W, Co)
    out = acc_ref[pl.ds(1, H), :, :] + bb_ref[0]
    o_ref[0] = jnp.where(out >= 0, out, LEAKY_SLOPE * out)


def kernel(feats, w0, scale0, bias0, w1, scale1, bias1, w2, scale2, bias2,
           w3, scale3, bias3, wb, b_scale, b_bias):
    sizes = (1, 2, 3, 6)
    N, Cin, H, W = feats.shape
    HW = H * W
    Cout = w0.shape[1]
    nCo = len(sizes) * Cout
    S = sum(s * s for s in sizes)
    Ct = nCo + Cin

    x_flat = jnp.transpose(feats, (0, 2, 3, 1)).reshape(N, HW, Cin)
    wb9 = wb.reshape(9, Ct, Cout)

    p_all, b_all, mask, mnot_first, mnot_last = _stage_constants(H, W, sizes, Cout)

    flops = 2 * N * (S * HW * Cin + S * Cin * nCo + S * nCo * 9 * Cout
                     + HW * S * 9 * Cout + 9 * HW * Cin * Cout)
    bytes_accessed = (4 * N * HW * Cin + 2 * S * HW + 2 * HW * S
                      + 4 * Ct * 9 * Cout + 4 * Cin * nCo
                      + 4 * N * HW * Cout)

    NB = 2
    NI = N // NB
    zero2 = lambda i, j: (0, 0)
    zero3 = lambda i, j: (0, 0, 0)

    out_nhwc = pl.pallas_call(
        _psp_kernel,
        out_shape=jax.ShapeDtypeStruct((N, H, W, Cout), jnp.float32),
        grid=(NB, NI),
        in_specs=[
            pl.BlockSpec((1, HW, Cin), lambda i, j: (i * NI + j, 0, 0)),
            pl.BlockSpec((S, HW), zero2),
            pl.BlockSpec((S, nCo), zero2),
            pl.BlockSpec((HW, S), zero2),
            pl.BlockSpec((HW, 1), zero2),
            pl.BlockSpec((HW, 1), zero2),
        ] + [pl.BlockSpec((Cin, Cout), zero2)] * 4
          + [pl.BlockSpec((1, Cout), zero2)] * 8
          + [
            pl.BlockSpec((9, Ct, Cout), zero3),
            pl.BlockSpec((1, Cout), zero2),
            pl.BlockSpec((1, Cout), zero2),
        ],
        out_specs=pl.BlockSpec((1, H, W, Cout), lambda i, j: (i * NI + j, 0, 0, 0)),
        scratch_shapes=[
            pltpu.VMEM((H + 2, W, Cout), jnp.float32),
            pltpu.VMEM((Ct, 9 * Cout), jnp.bfloat16),
            pltpu.VMEM((Cin, nCo), jnp.bfloat16),
        ],
        compiler_params=pltpu.CompilerParams(
            dimension_semantics=("parallel", "arbitrary"),
            vmem_limit_bytes=VMEM_LIMIT_BYTES),
        cost_estimate=pl.CostEstimate(flops=flops, transcendentals=0,
                                      bytes_accessed=bytes_accessed),
    )(x_flat, p_all, mask, b_all, mnot_first, mnot_last,
      w0, w1, w2, w3, scale0, scale1, scale2, scale3,
      bias0, bias1, bias2, bias3, wb9, b_scale, b_bias)

    return jnp.transpose(out_nhwc, (0, 3, 1, 2))
